# baseline (speedup 1.0000x reference)
import jax
import jax.numpy as jnp
from jax.experimental import pallas as pl
from jax.experimental.pallas import tpu as pltpu

_BI = 200


def _pass1_kernel(adj_a_ref, adj_b_ref, x_ref, w1_ref, b1_ref, w2_ref,
                  out_a_ref, out_b_ref):
    x = x_ref[...]
    w1 = w1_ref[...]
    b1 = b1_ref[...]
    w2 = w2_ref[...]
    for adj_ref, out_ref in ((adj_a_ref, out_a_ref), (adj_b_ref, out_b_ref)):
        adj = adj_ref[...]
        s = jnp.dot(adj, x, preferred_element_type=jnp.float32)
        r = jnp.sum(adj, axis=1, keepdims=True)
        h = jnp.maximum(
            jnp.dot(s, w1, preferred_element_type=jnp.float32) + r * b1,
            0.0,
        )
        out_ref[...] = jnp.dot(h, w2, preferred_element_type=jnp.float32)


def _pass2_kernel(adj_a_ref, adj_b_ref, b_ref, b2_ref, out_a_ref, out_b_ref):
    b = b_ref[...]
    b2 = b2_ref[...]
    for adj_ref, out_ref in ((adj_a_ref, out_a_ref), (adj_b_ref, out_b_ref)):
        adj = adj_ref[...]
        o = (
            jnp.dot(adj, b, preferred_element_type=jnp.float32)
            + jnp.sum(adj, axis=1, keepdims=True) * b2
        )
        nrm = jnp.sqrt(jnp.sum(o * o, axis=1, keepdims=True))
        out_ref[...] = o / jnp.maximum(nrm, 1e-12)


def kernel(x, Adj_, W1, b1, W2, b2):
    n, in_dim = x.shape
    emb_dim = W2.shape[1]
    b1r = b1.reshape(1, -1)
    b2r = b2.reshape(1, -1)
    grid = (n // (2 * _BI),)
    cparams = pltpu.CompilerParams(
        dimension_semantics=("arbitrary",),
    )
    adj_a_spec = pl.BlockSpec((_BI, n), lambda i: (2 * i, 0))
    adj_b_spec = pl.BlockSpec((_BI, n), lambda i: (2 * i + 1, 0))
    out_a_spec = pl.BlockSpec((_BI, emb_dim), lambda i: (2 * i, 0))
    out_b_spec = pl.BlockSpec((_BI, emb_dim), lambda i: (2 * i + 1, 0))
    out_sds = jax.ShapeDtypeStruct((n, emb_dim), jnp.float32)

    B, B_dup = pl.pallas_call(
        _pass1_kernel,
        grid=grid,
        in_specs=[
            adj_a_spec,
            adj_b_spec,
            pl.BlockSpec((n, in_dim), lambda i: (0, 0)),
            pl.BlockSpec(W1.shape, lambda i: (0, 0)),
            pl.BlockSpec(b1r.shape, lambda i: (0, 0)),
            pl.BlockSpec(W2.shape, lambda i: (0, 0)),
        ],
        out_specs=[out_a_spec, out_b_spec],
        out_shape=[out_sds, out_sds],
        compiler_params=cparams,
    )(Adj_, Adj_, x, W1, b1r, W2)
    B = jnp.where(jnp.arange(n)[:, None] % (2 * _BI) < _BI, B, B_dup)

    emb, emb_dup = pl.pallas_call(
        _pass2_kernel,
        grid=grid,
        in_specs=[
            adj_a_spec,
            adj_b_spec,
            pl.BlockSpec((n, emb_dim), lambda i: (0, 0)),
            pl.BlockSpec(b2r.shape, lambda i: (0, 0)),
        ],
        out_specs=[out_a_spec, out_b_spec],
        out_shape=[out_sds, out_sds],
        compiler_params=cparams,
    )(Adj_, Adj_, B, b2r)
    emb = jnp.where(jnp.arange(n)[:, None] % (2 * _BI) < _BI, emb, emb_dup)

    return emb


# speedup vs baseline: 1.1519x; 1.1519x over previous
import jax
import jax.numpy as jnp
from jax.experimental import pallas as pl
from jax.experimental.pallas import tpu as pltpu

_BI = 400


def _fused_kernel(adj_ref, x_ref, w1_ref, b1_ref, w2_ref, b2_ref,
                  out_ref, bbuf_ref):
    i = pl.program_id(0)
    ns = pl.num_programs(0) // 2
    strip = jax.lax.rem(i, ns)
    adj = adj_ref[...]
    r = jnp.sum(adj, axis=1, keepdims=True)

    @pl.when(i < ns)
    def _():
        s = jnp.dot(adj, x_ref[...], preferred_element_type=jnp.float32)
        h = jnp.maximum(
            jnp.dot(s, w1_ref[...], preferred_element_type=jnp.float32)
            + r * b1_ref[...],
            0.0,
        )
        bbuf_ref[pl.ds(strip * _BI, _BI), :] = jnp.dot(
            h, w2_ref[...], preferred_element_type=jnp.float32
        )

    @pl.when(i >= ns)
    def _():
        o = (
            jnp.dot(adj, bbuf_ref[...], preferred_element_type=jnp.float32)
            + r * b2_ref[...]
        )
        nrm = jnp.sqrt(jnp.sum(o * o, axis=1, keepdims=True))
        out_ref[...] = o / jnp.maximum(nrm, 1e-12)


def kernel(x, Adj_, W1, b1, W2, b2):
    n, in_dim = x.shape
    emb_dim = W2.shape[1]
    b1r = b1.reshape(1, -1)
    b2r = b2.reshape(1, -1)
    ns = n // _BI
    cparams = pltpu.CompilerParams(
        dimension_semantics=("arbitrary",),
    )

    emb = pl.pallas_call(
        _fused_kernel,
        grid=(2 * ns,),
        in_specs=[
            pl.BlockSpec((_BI, n), lambda i: (jax.lax.rem(i, n // _BI), 0)),
            pl.BlockSpec((n, in_dim), lambda i: (0, 0)),
            pl.BlockSpec(W1.shape, lambda i: (0, 0)),
            pl.BlockSpec(b1r.shape, lambda i: (0, 0)),
            pl.BlockSpec(W2.shape, lambda i: (0, 0)),
            pl.BlockSpec(b2r.shape, lambda i: (0, 0)),
        ],
        out_specs=pl.BlockSpec((_BI, emb_dim), lambda i: (jax.lax.rem(i, n // _BI), 0)),
        out_shape=jax.ShapeDtypeStruct((n, emb_dim), jnp.float32),
        scratch_shapes=[pltpu.VMEM((n, emb_dim), jnp.float32)],
        compiler_params=cparams,
    )(Adj_, x, W1, b1r, W2, b2r)

    return emb
